# baseline (speedup 1.0000x reference)
import functools

import jax
import jax.numpy as jnp
import numpy as np
from jax import lax
from jax.experimental import pallas as pl
from jax.experimental.pallas import tpu as pltpu
from jax.experimental.pallas import tpu_sc as plsc

_BATCH = 4096
_SEQ = 200
_DIM = 64
_CLS = 50
_NW = 32
_BPW = _BATCH // _NW
_NBUF = 4
_NCHUNK = _BPW // _NBUF
_SPLITS = ((0, 104), (104, 96))
_UNROLL = 8

_PERM = np.array(
    [32 * (p // 32) + 2 * (p % 16) + ((p // 16) % 2) for p in range(_DIM)],
    dtype=np.int32,
)


def _pool_body(x_hbm, tbl_hbm, out_hbm, idx_v, rows_v, pool_v, sems):
    wid = lax.axis_index("s") * 2 + lax.axis_index("c")
    base = wid * _BPW
    pltpu.sync_copy(x_hbm.at[pl.ds(base * _SEQ, _BPW * _SEQ)], idx_v)

    def issue(b, j):
        for (o, n) in _SPLITS:
            pltpu.async_copy(
                tbl_hbm.at[idx_v.at[pl.ds(b * _SEQ + o, n)]],
                rows_v.at[j, pl.ds(o, n)],
                sems.at[j],
            )

    def wait(j):
        pltpu.make_async_copy(
            tbl_hbm.at[pl.ds(0, _SEQ)], rows_v.at[j], sems.at[j]
        ).wait()

    def accumulate(b, j):
        zero = jnp.zeros((16,), jnp.float32)

        def body(i, accs):
            accs = list(accs)
            for u in range(_UNROLL):
                s = i * _UNROLL + u
                for h in range(2):
                    w = plsc.bitcast(rows_v[j, s, pl.ds(h * 32, 32)], jnp.int32)
                    lo = plsc.bitcast(lax.shift_left(w, 16), jnp.float32)
                    hi = plsc.bitcast(
                        lax.bitwise_and(w, jnp.int32(-65536)), jnp.float32
                    )
                    accs[2 * h] = accs[2 * h] + lo
                    accs[2 * h + 1] = accs[2 * h + 1] + hi
            return tuple(accs)

        accs = lax.fori_loop(0, _SEQ // _UNROLL, body, (zero,) * 4)
        for d in range(4):
            pool_v[b, pl.ds(d * 16, 16)] = accs[d]

    for j in range(_NBUF):
        issue(j, j)

    def chunk(t, carry):
        for j in range(_NBUF):
            b = t * _NBUF + j
            wait(j)
            accumulate(b, j)
            issue(b + _NBUF, j)
        return carry

    lax.fori_loop(0, _NCHUNK - 1, chunk, 0)
    for j in range(_NBUF):
        wait(j)
        accumulate((_NCHUNK - 1) * _NBUF + j, j)

    pltpu.sync_copy(pool_v, out_hbm.at[pl.ds(base, _BPW)])


_pool = functools.partial(
    pl.kernel,
    out_type=jax.ShapeDtypeStruct((_BATCH, _DIM), jnp.float32),
    mesh=plsc.VectorSubcoreMesh(core_axis_name="c", subcore_axis_name="s"),
    scratch_types=[
        pltpu.VMEM((_BPW * _SEQ,), jnp.int32),
        pltpu.VMEM((_NBUF, _SEQ, _DIM), jnp.bfloat16),
        pltpu.VMEM((_BPW, _DIM), jnp.float32),
        pltpu.SemaphoreType.DMA((_NBUF,)),
    ],
    compiler_params=pltpu.CompilerParams(
        use_tc_tiling_on_sc=False, needs_layout_passes=False
    ),
)(_pool_body)


def _head_body(p_ref, w_ref, b_ref, o_ref):
    o_ref[...] = (
        jnp.dot(p_ref[...], w_ref[...], preferred_element_type=jnp.float32)
        + b_ref[...]
    )


def kernel(x, emb_table, W, b):
    x = x.astype(jnp.int32).reshape(-1)
    tbl_bf = (emb_table * jnp.float32(1.0 / _SEQ)).astype(jnp.bfloat16)
    pooled = _pool(x, tbl_bf)
    out = pl.pallas_call(
        _head_body,
        out_shape=jax.ShapeDtypeStruct((_BATCH, _CLS), jnp.float32),
    )(pooled, W[_PERM], b.reshape(1, _CLS))
    return out


# speedup vs baseline: 1.1187x; 1.1187x over previous
import functools

import jax
import jax.numpy as jnp
from jax import lax
from jax.experimental import pallas as pl
from jax.experimental.pallas import tpu as pltpu
from jax.experimental.pallas import tpu_sc as plsc

_BATCH = 4096
_SEQ = 200
_DIM = 64
_CLS = 50
_VROW = 128
_NW = 32
_BPW = _BATCH // _NW
_NBUF = 2
_NCHUNK = _BPW // _NBUF
_SPLITS = ((0, 104), (104, 96))


def _pool_body(x_hbm, tbl_hbm, split_hbm, out_hbm, idx_v, rows_v, pool_v,
               split_v, sems):
    wid = lax.axis_index("s") * 2 + lax.axis_index("c")
    base = wid * _BPW
    pltpu.sync_copy(x_hbm.at[pl.ds(base * _SEQ, _BPW * _SEQ)], idx_v)
    pltpu.sync_copy(split_hbm.at[pl.ds(base, _BPW)], split_v)

    def issue(b, j):
        for (o, n) in _SPLITS:
            pltpu.async_copy(
                tbl_hbm.at[idx_v.at[pl.ds(b * _SEQ + o, n)]],
                rows_v.at[j, pl.ds(o, n)],
                sems.at[j],
            )

    def wait(j):
        pltpu.make_async_copy(
            tbl_hbm.at[pl.ds(0, _SEQ)], rows_v.at[j], sems.at[j]
        ).wait()

    def accumulate(b, j):
        zero = jnp.zeros((16,), jnp.float32)
        sv = split_v[pl.ds((b // 16) * 16, 16)]
        lane = jnp.int32(b % 16)
        n_even = jnp.sum(jnp.where(lax.iota(jnp.int32, 16) == lane, sv, 0))

        def make_body(col):
            def body(s, accs):
                accs = list(accs)
                for d in range(4):
                    accs[d] = accs[d] + rows_v[j, s, pl.ds(col + d * 16, 16)]
                return tuple(accs)
            return body

        accs = lax.fori_loop(0, n_even, make_body(0), (zero,) * 4)
        accs = lax.fori_loop(n_even, _SEQ, make_body(_DIM), accs)
        scale = jnp.float32(1.0 / _SEQ)
        for d in range(4):
            pool_v[b, pl.ds(d * 16, 16)] = accs[d] * scale

    for j in range(_NBUF):
        issue(j, j)

    def chunk(t, carry):
        for j in range(_NBUF):
            b = t * _NBUF + j
            wait(j)
            accumulate(b, j)
            issue(b + _NBUF, j)
        return carry

    lax.fori_loop(0, _NCHUNK - 1, chunk, 0)
    for j in range(_NBUF):
        wait(j)
        accumulate((_NCHUNK - 1) * _NBUF + j, j)

    pltpu.sync_copy(pool_v, out_hbm.at[pl.ds(base, _BPW)])


_pool = functools.partial(
    pl.kernel,
    out_type=jax.ShapeDtypeStruct((_BATCH, _DIM), jnp.float32),
    mesh=plsc.VectorSubcoreMesh(core_axis_name="c", subcore_axis_name="s"),
    scratch_types=[
        pltpu.VMEM((_BPW * _SEQ,), jnp.int32),
        pltpu.VMEM((_NBUF, _SEQ, _VROW), jnp.float32),
        pltpu.VMEM((_BPW, _DIM), jnp.float32),
        pltpu.VMEM((_BPW,), jnp.int32),
        pltpu.SemaphoreType.DMA((_NBUF,)),
    ],
    compiler_params=pltpu.CompilerParams(needs_layout_passes=False),
)(_pool_body)


def _head_body(p_ref, w_ref, b_ref, o_ref):
    o_ref[...] = (
        jnp.dot(p_ref[...], w_ref[...], preferred_element_type=jnp.float32)
        + b_ref[...]
    )


def kernel(x, emb_table, W, b):
    x = x.astype(jnp.int32)
    par = x & 1
    order = jnp.argsort(par, axis=1, stable=True)
    xs = jnp.take_along_axis(x, order, axis=1)
    split = (_SEQ - par.sum(axis=1)).astype(jnp.int32)
    x2 = (xs >> 1).reshape(-1)
    tblv = emb_table.reshape(500000, _VROW)
    pooled = _pool(x2, tblv, split)
    out = pl.pallas_call(
        _head_body,
        out_shape=jax.ShapeDtypeStruct((_BATCH, _CLS), jnp.float32),
    )(pooled, W, b.reshape(1, _CLS))
    return out


# speedup vs baseline: 1.5427x; 1.3790x over previous
import functools

import jax
import jax.numpy as jnp
from jax import lax
from jax.experimental import pallas as pl
from jax.experimental.pallas import tpu as pltpu
from jax.experimental.pallas import tpu_sc as plsc

_BATCH = 4096
_SEQ = 200
_DIM = 64
_CLS = 50
_NW = 32
_BPW = _BATCH // _NW
_NBUF = 4
_NCHUNK = _BPW // _NBUF
_SPLITS = ((0, 104), (104, 96))
_UNROLL = 8


def _pool_body(x_hbm, tbl_hbm, out_hbm, idx_v, rows_v, pool_v, sems):
    wid = lax.axis_index("s") * 2 + lax.axis_index("c")
    base = wid * _BPW
    pltpu.sync_copy(x_hbm.at[pl.ds(base * _SEQ, _BPW * _SEQ)], idx_v)

    def issue(b, j):
        for (o, n) in _SPLITS:
            pltpu.async_copy(
                tbl_hbm.at[idx_v.at[pl.ds(b * _SEQ + o, n)]],
                rows_v.at[j, pl.ds(o, n)],
                sems.at[j],
            )

    def wait(j):
        pltpu.make_async_copy(
            tbl_hbm.at[pl.ds(0, _SEQ)], rows_v.at[j], sems.at[j]
        ).wait()

    def accumulate(b, j):
        zero = jnp.zeros((16,), jnp.float32)

        def body(i, accs):
            accs = list(accs)
            for u in range(_UNROLL):
                s = i * _UNROLL + u
                for d in range(4):
                    accs[d] = accs[d] + rows_v[j, s, pl.ds(d * 16, 16)]
            return tuple(accs)

        accs = lax.fori_loop(0, _SEQ // _UNROLL, body, (zero,) * 4)
        scale = jnp.float32(1.0 / _SEQ)
        for d in range(4):
            pool_v[b, pl.ds(d * 16, 16)] = accs[d] * scale

    for j in range(_NBUF):
        issue(j, j)

    def chunk(t, carry):
        for j in range(_NBUF):
            b = t * _NBUF + j
            wait(j)
            accumulate(b, j)
            issue(b + _NBUF, j)
        return carry

    lax.fori_loop(0, _NCHUNK - 1, chunk, 0)
    for j in range(_NBUF):
        wait(j)
        accumulate((_NCHUNK - 1) * _NBUF + j, j)

    pltpu.sync_copy(pool_v, out_hbm.at[pl.ds(base, _BPW)])


_pool = functools.partial(
    pl.kernel,
    out_type=jax.ShapeDtypeStruct((_BATCH, _DIM), jnp.float32),
    mesh=plsc.VectorSubcoreMesh(core_axis_name="c", subcore_axis_name="s"),
    scratch_types=[
        pltpu.VMEM((_BPW * _SEQ,), jnp.int32),
        pltpu.VMEM((_NBUF, _SEQ, _DIM), jnp.float32),
        pltpu.VMEM((_BPW, _DIM), jnp.float32),
        pltpu.SemaphoreType.DMA((_NBUF,)),
    ],
    compiler_params=pltpu.CompilerParams(use_tc_tiling_on_sc=False),
)(_pool_body)


def _head_body(p_ref, w_ref, b_ref, o_ref):
    o_ref[...] = (
        jnp.dot(p_ref[...], w_ref[...], preferred_element_type=jnp.float32)
        + b_ref[...]
    )


def kernel(x, emb_table, W, b):
    x = x.astype(jnp.int32).reshape(-1)
    pooled = _pool(x, emb_table)
    out = pl.pallas_call(
        _head_body,
        out_shape=jax.ShapeDtypeStruct((_BATCH, _CLS), jnp.float32),
    )(pooled, W, b.reshape(1, _CLS))
    return out
